# Initial kernel scaffold; baseline (speedup 1.0000x reference)
import jax
import jax.numpy as jnp
from jax.experimental import pallas as pl


def kernel(x, w_router, w_in, w_out, bias):
    raise NotImplementedError("write your pallas kernel here")


# speedup vs baseline: 3.4256x; 3.4256x over previous
import functools

import jax
import jax.numpy as jnp
from jax.experimental import pallas as pl
from jax.experimental.pallas import tpu as pltpu

_BM = 512


def _gmm_body(em, tm, vm, se, ee, x_ref, win_ref, wout_ref, gates_ref, out_ref):
    i = pl.program_id(0)
    e = em[i]
    t = tm[i]
    valid = vm[i]
    bm, d = x_ref.shape
    h2 = win_ref.shape[2]
    h = h2 // 2

    @pl.when(valid == 1)
    def _():
        rows = t * bm + jax.lax.broadcasted_iota(jnp.int32, (bm, 1), 0)
        mask = (rows >= se[e]) & (rows < ee[e])
        hh = jnp.dot(x_ref[...], win_ref[0], preferred_element_type=jnp.float32)
        h1 = hh[:, :h]
        hg = hh[:, h:]
        act = h1 * jax.nn.sigmoid(h1) * hg
        o = jnp.dot(act, wout_ref[0], preferred_element_type=jnp.float32)
        o = o * gates_ref[...]
        out_ref[...] = jnp.where(mask, o, out_ref[...])


def _grouped_ffn(x_sorted, w_in, w_out, gates_sorted, starts, ends):
    tk, d = x_sorted.shape
    e_num, _, h2 = w_in.shape
    h = h2 // 2
    m_tiles = tk // _BM
    max_steps = m_tiles + e_num - 1

    counts = ends - starts
    tile_lo = starts // _BM
    tile_hi = (ends + _BM - 1) // _BM
    ntiles = jnp.where(counts > 0, tile_hi - tile_lo, 0)
    cum = jnp.cumsum(ntiles)
    total = cum[-1]
    first_step = cum - ntiles

    steps = jnp.arange(max_steps, dtype=jnp.int32)
    e_of = jnp.searchsorted(cum, steps, side="right").astype(jnp.int32)
    valid = (steps < total).astype(jnp.int32)
    e_last = jnp.searchsorted(cum, total - 1, side="right").astype(jnp.int32)
    e_of = jnp.where(valid == 1, jnp.minimum(e_of, e_num - 1), e_last)
    t_of = jnp.where(
        valid == 1,
        tile_lo[e_of] + steps - first_step[e_of],
        m_tiles - 1,
    ).astype(jnp.int32)

    grid_spec = pltpu.PrefetchScalarGridSpec(
        num_scalar_prefetch=5,
        grid=(max_steps,),
        in_specs=[
            pl.BlockSpec((_BM, d), lambda i, em, tm, vm, se, ee: (tm[i], 0)),
            pl.BlockSpec((1, d, h2), lambda i, em, tm, vm, se, ee: (em[i], 0, 0)),
            pl.BlockSpec((1, h, d), lambda i, em, tm, vm, se, ee: (em[i], 0, 0)),
            pl.BlockSpec((_BM, 1), lambda i, em, tm, vm, se, ee: (tm[i], 0)),
        ],
        out_specs=pl.BlockSpec((_BM, d), lambda i, em, tm, vm, se, ee: (tm[i], 0)),
    )
    return pl.pallas_call(
        _gmm_body,
        grid_spec=grid_spec,
        out_shape=jax.ShapeDtypeStruct((tk, d), jnp.float32),
        compiler_params=pltpu.CompilerParams(
            dimension_semantics=("arbitrary",),
            vmem_limit_bytes=100 * 1024 * 1024,
        ),
    )(
        e_of,
        t_of,
        valid,
        starts.astype(jnp.int32),
        ends.astype(jnp.int32),
        x_sorted,
        w_in,
        w_out,
        gates_sorted[:, None],
    )


@jax.jit
def kernel(x, w_router, w_in, w_out, bias):
    bsz, length, d = x.shape
    e_num = w_router.shape[1]
    k = 2
    xf = x.reshape(-1, d)
    t = xf.shape[0]

    logits = (xf @ w_router).astype(jnp.float32)
    top_k_logits, top_k_indices = jax.lax.top_k(logits, k)
    top_k_gates = jax.nn.softmax(top_k_logits, axis=1).astype(x.dtype)
    probs = jax.nn.softmax(logits, axis=1)
    probs_sum = probs.sum(axis=0)
    freq = jnp.zeros((e_num,), jnp.float32).at[top_k_indices.reshape(-1)].add(
        (top_k_gates.reshape(-1) > 0).astype(jnp.float32)
    )
    lsesq = (jax.nn.logsumexp(logits, axis=-1) ** 2).sum()
    probs_normalized = probs_sum / jnp.sum(probs_sum)
    freq_normalized = freq / jnp.sum(freq)
    switchloss = e_num * (probs_normalized * freq_normalized).sum()
    zloss = lsesq / t
    loss = switchloss + 0.1 * zloss

    flat_experts = top_k_indices.reshape(-1)
    ise = jnp.argsort(flat_experts)
    batch_index = ise // k
    gates_sorted = top_k_gates.reshape(-1)[ise]

    counts = jnp.zeros((e_num,), jnp.int32).at[flat_experts].add(1)
    ends = jnp.cumsum(counts).astype(jnp.int32)
    starts = ends - counts

    x_sorted = xf[batch_index]
    out_w = _grouped_ffn(x_sorted, w_in, w_out, gates_sorted, starts, ends)

    y = jnp.zeros((t, d), jnp.float32).at[batch_index].add(out_w)
    y = y.reshape(bsz, length, d) + bias
    return (y, loss)
